# baseline (speedup 1.0000x reference)
import functools

import jax
import jax.numpy as jnp
import numpy as np
from jax import lax
from jax.experimental import pallas as pl
from jax.experimental.pallas import tpu as pltpu
from jax.experimental.pallas import tpu_sc as plsc

N = 10000
F = 128
E = 320000
N_PAD = 10240
ROWS_PER_TILE = N_PAD // 16
E_ROWS = 2560
E_PAD = E_ROWS * 128
TILE_EROWS = E_ROWS // 32
SW = 16
N_SW = TILE_EROWS // SW

_PAD = np.arange(E_PAD - E, dtype=np.int32)
_PAD_SRC = jnp.asarray((_PAD * 131) % N, dtype=jnp.int32)
_PAD_DST = jnp.asarray(N + _PAD % (N_PAD - N), dtype=jnp.int32)

_mesh = plsc.VectorSubcoreMesh(core_axis_name="c", subcore_axis_name="s")


def _zero_rows(buf, nrows):
    @pl.loop(0, nrows)
    def _(i):
        for j in range(8):
            buf[i, pl.ds(j * 16, 16)] = jnp.zeros((16,), jnp.float32)


@functools.partial(
    pl.kernel,
    out_type=jax.ShapeDtypeStruct((2, N_PAD), jnp.float32),
    mesh=_mesh,
    scratch_types=[
        pltpu.VMEM((TILE_EROWS, 128), jnp.int32),
        pltpu.VMEM((2, 128), jnp.float32),
        pltpu.VMEM_SHARED((N_PAD,), jnp.float32),
        pltpu.SemaphoreType.DMA,
    ],
)
def _deg_kernel(dst_hbm, out_hbm, dst_v, const_v, acc_sh, sem_i):
    cid = lax.axis_index("c")
    sid = lax.axis_index("s")
    wid = cid * 16 + sid
    eb = wid * TILE_EROWS * 128

    @pl.loop(0, TILE_EROWS)
    def _(w):
        pltpu.async_copy(dst_hbm.at[pl.ds(eb + w * 128, 128)], dst_v.at[w],
                         sem_i)

    for j in range(8):
        const_v[0, pl.ds(j * 16, 16)] = jnp.ones((16,), jnp.float32)
        const_v[1, pl.ds(j * 16, 16)] = jnp.zeros((16,), jnp.float32)
    base = sid * ROWS_PER_TILE
    for k in range(ROWS_PER_TILE // 128):
        pltpu.sync_copy(const_v.at[1], acc_sh.at[pl.ds(base + k * 128, 128)])

    @pl.loop(0, TILE_EROWS)
    def _(w):
        pltpu.make_async_copy(dst_hbm.at[pl.ds(0, 128)], dst_v.at[w],
                              sem_i).wait()

    plsc.subcore_barrier()

    @pl.loop(0, TILE_EROWS)
    def _(w):
        pltpu.sync_copy(const_v.at[0], acc_sh.at[dst_v.at[w]], add=True)

    plsc.subcore_barrier()
    pltpu.sync_copy(acc_sh.at[pl.ds(base, ROWS_PER_TILE)],
                    out_hbm.at[cid, pl.ds(base, ROWS_PER_TILE)])


@functools.partial(
    pl.kernel,
    out_type=jax.ShapeDtypeStruct((2, N_PAD, F), jnp.float32),
    mesh=_mesh,
    scratch_types=[
        pltpu.VMEM((2, SW * 128), jnp.int32),
        pltpu.VMEM((2, SW, 128), jnp.int32),
        pltpu.VMEM((2, 128, F), jnp.float32),
        pltpu.VMEM_SHARED((N_PAD, F), jnp.float32),
        pltpu.SemaphoreType.DMA,
        pltpu.SemaphoreType.DMA,
        pltpu.SemaphoreType.DMA,
        pltpu.SemaphoreType.DMA,
    ],
)
def _edge_kernel(u_hbm, src_hbm, dst_hbm, out_hbm, src_v, dst_v, rows_v,
                 acc_sh, sem_g0, sem_g1, sem_i0, sem_i1):
    cid = lax.axis_index("c")
    sid = lax.axis_index("s")
    wid = cid * 16 + sid
    eb = wid * TILE_EROWS * 128
    SWE = SW * 128
    gsems = [sem_g0, sem_g1]
    isems = [sem_i0, sem_i1]

    def _fetch_idx(off, slot, isem):
        pltpu.async_copy(src_hbm.at[pl.ds(off, SWE)], src_v.at[slot], isem)
        for j in range(SW):
            pltpu.async_copy(dst_hbm.at[pl.ds(off + j * 128, 128)],
                             dst_v.at[slot, j], isem)

    def _wait_idx(slot, isem):
        pltpu.make_async_copy(src_hbm.at[pl.ds(0, SWE)], src_v.at[slot],
                              isem).wait()
        for j in range(SW):
            pltpu.make_async_copy(dst_hbm.at[pl.ds(0, 128)],
                                  dst_v.at[slot, j], isem).wait()

    _fetch_idx(eb, 0, sem_i0)
    _zero_rows(rows_v.at[0], 128)
    base = sid * ROWS_PER_TILE
    for k in range(ROWS_PER_TILE // 128):
        pltpu.sync_copy(rows_v.at[0], acc_sh.at[pl.ds(base + k * 128, 128)])
    plsc.subcore_barrier()

    def _do_sw(sw, slot):
        nslot = 1 - slot
        nr0 = eb + jnp.where(sw + 1 < N_SW, (sw + 1) * SWE, 0)
        _fetch_idx(nr0, nslot, isems[nslot])
        _wait_idx(slot, isems[slot])
        for j in range(SW):
            b = j % 2
            pltpu.async_copy(
                u_hbm.at[src_v.at[slot, pl.ds(j * 128, 128)]], rows_v.at[b],
                gsems[b])
            if j:
                pb = 1 - b
                pltpu.make_async_copy(
                    u_hbm.at[src_v.at[slot, pl.ds((j - 1) * 128, 128)]],
                    rows_v.at[pb], gsems[pb]).wait()
                pltpu.sync_copy(rows_v.at[pb],
                                acc_sh.at[dst_v.at[slot, j - 1]], add=True)
        lb = (SW - 1) % 2
        pltpu.make_async_copy(
            u_hbm.at[src_v.at[slot, pl.ds((SW - 1) * 128, 128)]],
            rows_v.at[lb], gsems[lb]).wait()
        pltpu.sync_copy(rows_v.at[lb], acc_sh.at[dst_v.at[slot, SW - 1]],
                        add=True)

    @pl.loop(0, N_SW - 1, step=2)
    def _(sw):
        _do_sw(sw, 0)
        _do_sw(sw + 1, 1)

    _do_sw(N_SW - 1, 0)

    _wait_idx(1, sem_i1)
    plsc.subcore_barrier()
    pltpu.sync_copy(acc_sh.at[pl.ds(base, ROWS_PER_TILE)],
                    out_hbm.at[cid, pl.ds(base, ROWS_PER_TILE)])


BM = 1024
GRID = N_PAD // BM
_HI = lax.Precision.HIGHEST


def _g(deg_ref):
    return lax.rsqrt(deg_ref[0, :] + deg_ref[1, :] + 1.0)[:, None]


def _proj_body(x_ref, win_ref, bin_ref, w1_ref, deg_ref, o_ref):
    w = jnp.dot(win_ref[...], w1_ref[...], preferred_element_type=jnp.float32,
                precision=_HI)
    c = jnp.dot(bin_ref[...], w1_ref[...], preferred_element_type=jnp.float32,
                precision=_HI)
    o_ref[...] = _g(deg_ref) * (
        jnp.dot(x_ref[...], w, preferred_element_type=jnp.float32,
                precision=_HI) + c)


_proj = pl.pallas_call(
    _proj_body,
    grid=(GRID,),
    in_specs=[pl.BlockSpec((BM, F), lambda r: (r, 0)),
              pl.BlockSpec((F, F), lambda r: (0, 0)),
              pl.BlockSpec((1, F), lambda r: (0, 0)),
              pl.BlockSpec((F, F), lambda r: (0, 0)),
              pl.BlockSpec((2, BM), lambda r: (0, r))],
    out_specs=pl.BlockSpec((BM, F), lambda r: (r, 0)),
    out_shape=jax.ShapeDtypeStruct((N_PAD, F), jnp.float32),
)


def _layer_body(sa_ref, sb_ref, u_ref, deg_ref, b_ref, w_ref, o_ref):
    g = _g(deg_ref)
    h = jnp.maximum(
        g * (sa_ref[0] + sb_ref[0] + u_ref[...]) + b_ref[...], 0.0)
    o_ref[...] = g * jnp.dot(h, w_ref[...], preferred_element_type=jnp.float32,
                             precision=_HI)


_layer = pl.pallas_call(
    _layer_body,
    grid=(GRID,),
    in_specs=[pl.BlockSpec((1, BM, F), lambda r: (0, r, 0)),
              pl.BlockSpec((1, BM, F), lambda r: (1, r, 0)),
              pl.BlockSpec((BM, F), lambda r: (r, 0)),
              pl.BlockSpec((2, BM), lambda r: (0, r)),
              pl.BlockSpec((1, F), lambda r: (0, 0)),
              pl.BlockSpec((F, F), lambda r: (0, 0))],
    out_specs=pl.BlockSpec((BM, F), lambda r: (r, 0)),
    out_shape=jax.ShapeDtypeStruct((N_PAD, F), jnp.float32),
)


def _final_body(sa_ref, sb_ref, u_ref, deg_ref, b_ref, o_ref):
    g = _g(deg_ref)
    o_ref[...] = jnp.maximum(
        g * (sa_ref[0] + sb_ref[0] + u_ref[...]) + b_ref[...], 0.0)


_final = pl.pallas_call(
    _final_body,
    grid=(GRID,),
    in_specs=[pl.BlockSpec((1, BM, F), lambda r: (0, r, 0)),
              pl.BlockSpec((1, BM, F), lambda r: (1, r, 0)),
              pl.BlockSpec((BM, F), lambda r: (r, 0)),
              pl.BlockSpec((2, BM), lambda r: (0, r)),
              pl.BlockSpec((1, F), lambda r: (0, 0))],
    out_specs=pl.BlockSpec((BM, F), lambda r: (r, 0)),
    out_shape=jax.ShapeDtypeStruct((N, F), jnp.float32),
)


def kernel(x, edge_index, W_in, b_in, W1, b1, W2, b2):
    src = edge_index[0].astype(jnp.int32)
    dst = edge_index[1].astype(jnp.int32)
    src_p = jnp.concatenate([src, _PAD_SRC])
    dst_p = jnp.concatenate([dst, _PAD_DST])

    deg = _deg_kernel(dst_p)
    u1 = _proj(x, W_in, b_in.reshape(1, F), W1, deg)
    s1 = _edge_kernel(u1, src_p, dst_p)
    u2 = _layer(s1, s1, u1, deg, b1.reshape(1, F), W2)
    s2 = _edge_kernel(u2, src_p, dst_p)
    return _final(s2, s2, u2, deg, b2.reshape(1, F))


# speedup vs baseline: 1.0449x; 1.0449x over previous
import functools

import jax
import jax.numpy as jnp
import numpy as np
from jax import lax
from jax.experimental import pallas as pl
from jax.experimental.pallas import tpu as pltpu
from jax.experimental.pallas import tpu_sc as plsc

N = 10000
F = 128
E = 320000
N_PAD = 10240
ROWS_PER_TILE = N_PAD // 16
E_ROWS = 2560
E_PAD = E_ROWS * 128
TILE_EROWS = E_ROWS // 32
SW = 16
N_SW = TILE_EROWS // SW

R_TOTAL = E // 128
LAST_ROWS = R_TOTAL - 31 * TILE_EROWS
LAST_TAIL = LAST_ROWS - SW

_mesh = plsc.VectorSubcoreMesh(core_axis_name="c", subcore_axis_name="s")


def _zero_rows(buf, nrows):
    @pl.loop(0, nrows)
    def _(i):
        for j in range(8):
            buf[i, pl.ds(j * 16, 16)] = jnp.zeros((16,), jnp.float32)


@functools.partial(
    pl.kernel,
    out_type=jax.ShapeDtypeStruct((2, N_PAD), jnp.float32),
    mesh=_mesh,
    scratch_types=[
        pltpu.VMEM((TILE_EROWS, 128), jnp.int32),
        pltpu.VMEM((2, 128), jnp.float32),
        pltpu.VMEM_SHARED((N_PAD,), jnp.float32),
        pltpu.SemaphoreType.DMA,
    ],
)
def _deg_kernel(ei_hbm, out_hbm, dst_v, const_v, acc_sh, sem_i):
    cid = lax.axis_index("c")
    sid = lax.axis_index("s")
    wid = cid * 16 + sid
    r0 = wid * TILE_EROWS
    nw = jnp.where(wid < 31, TILE_EROWS, LAST_ROWS)

    @pl.loop(0, TILE_EROWS)
    def _(w):
        @pl.when(w < nw)
        def _():
            pltpu.async_copy(ei_hbm.at[1, pl.ds((r0 + w) * 128, 128)],
                             dst_v.at[w], sem_i)

    for j in range(8):
        const_v[0, pl.ds(j * 16, 16)] = jnp.ones((16,), jnp.float32)
        const_v[1, pl.ds(j * 16, 16)] = jnp.zeros((16,), jnp.float32)
    base = sid * ROWS_PER_TILE
    for k in range(ROWS_PER_TILE // 128):
        pltpu.sync_copy(const_v.at[1], acc_sh.at[pl.ds(base + k * 128, 128)])

    @pl.loop(0, TILE_EROWS)
    def _(w):
        @pl.when(w < nw)
        def _():
            pltpu.make_async_copy(ei_hbm.at[1, pl.ds(0, 128)], dst_v.at[w],
                                  sem_i).wait()

    plsc.subcore_barrier()

    @pl.loop(0, TILE_EROWS)
    def _(w):
        @pl.when(w < nw)
        def _():
            pltpu.sync_copy(const_v.at[0], acc_sh.at[dst_v.at[w]], add=True)

    plsc.subcore_barrier()
    pltpu.sync_copy(acc_sh.at[pl.ds(base, ROWS_PER_TILE)],
                    out_hbm.at[cid, pl.ds(base, ROWS_PER_TILE)])


@functools.partial(
    pl.kernel,
    out_type=jax.ShapeDtypeStruct((2, N_PAD, F), jnp.float32),
    mesh=_mesh,
    scratch_types=[
        pltpu.VMEM((2, SW, 128), jnp.int32),
        pltpu.VMEM((2, SW, 128), jnp.int32),
        pltpu.VMEM((2, 128, F), jnp.float32),
        pltpu.VMEM_SHARED((N_PAD, F), jnp.float32),
        pltpu.SemaphoreType.DMA,
        pltpu.SemaphoreType.DMA,
        pltpu.SemaphoreType.DMA,
        pltpu.SemaphoreType.DMA,
    ],
)
def _edge_kernel(u_hbm, ei_hbm, out_hbm, src_v, dst_v, rows_v,
                 acc_sh, sem_g0, sem_g1, sem_i0, sem_i1):
    cid = lax.axis_index("c")
    sid = lax.axis_index("s")
    wid = cid * 16 + sid
    r0 = wid * TILE_EROWS
    gsems = [sem_g0, sem_g1]
    isems = [sem_i0, sem_i1]

    def _fetch_idx(row0, slot, n, isem):
        for j in range(n):
            pltpu.async_copy(ei_hbm.at[0, pl.ds((row0 + j) * 128, 128)],
                             src_v.at[slot, j], isem)
            pltpu.async_copy(ei_hbm.at[1, pl.ds((row0 + j) * 128, 128)],
                             dst_v.at[slot, j], isem)

    def _wait_idx(slot, n, isem):
        for j in range(n):
            pltpu.make_async_copy(ei_hbm.at[0, pl.ds(0, 128)],
                                  src_v.at[slot, j], isem).wait()
            pltpu.make_async_copy(ei_hbm.at[1, pl.ds(0, 128)],
                                  dst_v.at[slot, j], isem).wait()

    def _do_windows(slot, n):
        pltpu.async_copy(u_hbm.at[src_v.at[slot, 0]], rows_v.at[0], gsems[0])
        for j in range(1, n):
            b = j % 2
            pltpu.async_copy(u_hbm.at[src_v.at[slot, j]], rows_v.at[b],
                             gsems[b])
            pb = 1 - b
            pltpu.make_async_copy(u_hbm.at[src_v.at[slot, j - 1]],
                                  rows_v.at[pb], gsems[pb]).wait()
            pltpu.sync_copy(rows_v.at[pb],
                            acc_sh.at[dst_v.at[slot, j - 1]], add=True)
        lb = (n - 1) % 2
        pltpu.make_async_copy(u_hbm.at[src_v.at[slot, n - 1]],
                              rows_v.at[lb], gsems[lb]).wait()
        pltpu.sync_copy(rows_v.at[lb], acc_sh.at[dst_v.at[slot, n - 1]],
                        add=True)

    @pl.when(wid < 31)
    def _():
        _fetch_idx(r0, 0, SW, sem_i0)

    @pl.when(wid == 31)
    def _():
        _fetch_idx(r0, 0, SW, sem_i0)
        _fetch_idx(r0 + SW, 1, LAST_TAIL, sem_i1)

    _zero_rows(rows_v.at[0], 128)
    base = sid * ROWS_PER_TILE
    for k in range(ROWS_PER_TILE // 128):
        pltpu.sync_copy(rows_v.at[0], acc_sh.at[pl.ds(base + k * 128, 128)])
    plsc.subcore_barrier()

    def _do_sw(sw, slot):
        nslot = 1 - slot
        nr0 = r0 + jnp.where(sw + 1 < N_SW, (sw + 1) * SW, 0)
        _fetch_idx(nr0, nslot, SW, isems[nslot])
        _wait_idx(slot, SW, isems[slot])
        _do_windows(slot, SW)

    @pl.when(wid < 31)
    def _():
        @pl.loop(0, N_SW - 1, step=2)
        def _(sw):
            _do_sw(sw, 0)
            _do_sw(sw + 1, 1)

        _do_sw(N_SW - 1, 0)
        _wait_idx(1, SW, sem_i1)

    @pl.when(wid == 31)
    def _():
        _wait_idx(0, SW, sem_i0)
        _do_windows(0, SW)
        _wait_idx(1, LAST_TAIL, sem_i1)
        _do_windows(1, LAST_TAIL)

    plsc.subcore_barrier()
    pltpu.sync_copy(acc_sh.at[pl.ds(base, ROWS_PER_TILE)],
                    out_hbm.at[cid, pl.ds(base, ROWS_PER_TILE)])


BM = 1024
GRID = N_PAD // BM
_HI = lax.Precision.HIGHEST


def _g(deg_ref):
    return lax.rsqrt(deg_ref[0, :] + deg_ref[1, :] + 1.0)[:, None]


def _proj_body(x_ref, win_ref, bin_ref, w1_ref, deg_ref, o_ref):
    w = jnp.dot(win_ref[...], w1_ref[...], preferred_element_type=jnp.float32,
                precision=_HI)
    c = jnp.dot(bin_ref[...], w1_ref[...], preferred_element_type=jnp.float32,
                precision=_HI)
    o_ref[...] = _g(deg_ref) * (
        jnp.dot(x_ref[...], w, preferred_element_type=jnp.float32,
                precision=_HI) + c)


_proj = pl.pallas_call(
    _proj_body,
    grid=(GRID,),
    in_specs=[pl.BlockSpec((BM, F), lambda r: (r, 0)),
              pl.BlockSpec((F, F), lambda r: (0, 0)),
              pl.BlockSpec((1, F), lambda r: (0, 0)),
              pl.BlockSpec((F, F), lambda r: (0, 0)),
              pl.BlockSpec((2, BM), lambda r: (0, r))],
    out_specs=pl.BlockSpec((BM, F), lambda r: (r, 0)),
    out_shape=jax.ShapeDtypeStruct((N_PAD, F), jnp.float32),
)


def _layer_body(sa_ref, sb_ref, u_ref, deg_ref, b_ref, w_ref, o_ref):
    g = _g(deg_ref)
    h = jnp.maximum(
        g * (sa_ref[0] + sb_ref[0] + u_ref[...]) + b_ref[...], 0.0)
    o_ref[...] = g * jnp.dot(h, w_ref[...], preferred_element_type=jnp.float32,
                             precision=_HI)


_layer = pl.pallas_call(
    _layer_body,
    grid=(GRID,),
    in_specs=[pl.BlockSpec((1, BM, F), lambda r: (0, r, 0)),
              pl.BlockSpec((1, BM, F), lambda r: (1, r, 0)),
              pl.BlockSpec((BM, F), lambda r: (r, 0)),
              pl.BlockSpec((2, BM), lambda r: (0, r)),
              pl.BlockSpec((1, F), lambda r: (0, 0)),
              pl.BlockSpec((F, F), lambda r: (0, 0))],
    out_specs=pl.BlockSpec((BM, F), lambda r: (r, 0)),
    out_shape=jax.ShapeDtypeStruct((N_PAD, F), jnp.float32),
)


def _final_body(sa_ref, sb_ref, u_ref, deg_ref, b_ref, o_ref):
    g = _g(deg_ref)
    o_ref[...] = jnp.maximum(
        g * (sa_ref[0] + sb_ref[0] + u_ref[...]) + b_ref[...], 0.0)


_final = pl.pallas_call(
    _final_body,
    grid=(GRID,),
    in_specs=[pl.BlockSpec((1, BM, F), lambda r: (0, r, 0)),
              pl.BlockSpec((1, BM, F), lambda r: (1, r, 0)),
              pl.BlockSpec((BM, F), lambda r: (r, 0)),
              pl.BlockSpec((2, BM), lambda r: (0, r)),
              pl.BlockSpec((1, F), lambda r: (0, 0))],
    out_specs=pl.BlockSpec((BM, F), lambda r: (r, 0)),
    out_shape=jax.ShapeDtypeStruct((N, F), jnp.float32),
)


def kernel(x, edge_index, W_in, b_in, W1, b1, W2, b2):
    ei = edge_index.astype(jnp.int32)

    deg = _deg_kernel(ei)
    u1 = _proj(x, W_in, b_in.reshape(1, F), W1, deg)
    s1 = _edge_kernel(u1, ei)
    u2 = _layer(s1, s1, u1, deg, b1.reshape(1, F), W2)
    s2 = _edge_kernel(u2, ei)
    return _final(s2, s2, u2, deg, b2.reshape(1, F))


# speedup vs baseline: 1.0587x; 1.0132x over previous
import functools

import jax
import jax.numpy as jnp
import numpy as np
from jax import lax
from jax.experimental import pallas as pl
from jax.experimental.pallas import tpu as pltpu
from jax.experimental.pallas import tpu_sc as plsc

N = 10000
F = 128
E = 320000
N_PAD = 10240
ROWS_PER_TILE = N_PAD // 16
E_ROWS = 2560
E_PAD = E_ROWS * 128
TILE_EROWS = E_ROWS // 32
SW = 16
N_SW = TILE_EROWS // SW

R_TOTAL = E // 128
LAST_ROWS = R_TOTAL - 31 * TILE_EROWS
LAST_TAIL = LAST_ROWS - SW

_mesh = plsc.VectorSubcoreMesh(core_axis_name="c", subcore_axis_name="s")


def _zero_rows(buf, nrows):
    @pl.loop(0, nrows)
    def _(i):
        for j in range(8):
            buf[i, pl.ds(j * 16, 16)] = jnp.zeros((16,), jnp.float32)


@functools.partial(
    pl.kernel,
    out_type=jax.ShapeDtypeStruct((2, N_PAD), jnp.float32),
    mesh=_mesh,
    scratch_types=[
        pltpu.VMEM((TILE_EROWS, 128), jnp.int32),
        pltpu.VMEM((2, 128), jnp.float32),
        pltpu.VMEM_SHARED((N_PAD,), jnp.float32),
        pltpu.SemaphoreType.DMA,
    ],
)
def _deg_kernel(ei_hbm, out_hbm, dst_v, const_v, acc_sh, sem_i):
    cid = lax.axis_index("c")
    sid = lax.axis_index("s")
    wid = cid * 16 + sid
    r0 = wid * TILE_EROWS
    nw = jnp.where(wid < 31, TILE_EROWS, LAST_ROWS)

    @pl.loop(0, TILE_EROWS)
    def _(w):
        @pl.when(w < nw)
        def _():
            pltpu.async_copy(ei_hbm.at[1, pl.ds((r0 + w) * 128, 128)],
                             dst_v.at[w], sem_i)

    for j in range(8):
        const_v[0, pl.ds(j * 16, 16)] = jnp.ones((16,), jnp.float32)
        const_v[1, pl.ds(j * 16, 16)] = jnp.zeros((16,), jnp.float32)
    base = sid * ROWS_PER_TILE
    for k in range(ROWS_PER_TILE // 128):
        pltpu.sync_copy(const_v.at[1], acc_sh.at[pl.ds(base + k * 128, 128)])

    @pl.loop(0, TILE_EROWS)
    def _(w):
        @pl.when(w < nw)
        def _():
            pltpu.make_async_copy(ei_hbm.at[1, pl.ds(0, 128)], dst_v.at[w],
                                  sem_i).wait()

    plsc.subcore_barrier()

    @pl.loop(0, TILE_EROWS)
    def _(w):
        @pl.when(w < nw)
        def _():
            pltpu.sync_copy(const_v.at[0], acc_sh.at[dst_v.at[w]], add=True)

    plsc.subcore_barrier()
    pltpu.sync_copy(acc_sh.at[pl.ds(base, ROWS_PER_TILE)],
                    out_hbm.at[cid, pl.ds(base, ROWS_PER_TILE)])


@functools.partial(
    pl.kernel,
    out_type=jax.ShapeDtypeStruct((2, N_PAD, F), jnp.float32),
    mesh=_mesh,
    scratch_types=[
        pltpu.VMEM((2, SW, 128), jnp.int32),
        pltpu.VMEM((2, SW, 128), jnp.int32),
        pltpu.VMEM((2, 128, F), jnp.float32),
        pltpu.VMEM_SHARED((N_PAD, F), jnp.float32),
        pltpu.SemaphoreType.DMA,
        pltpu.SemaphoreType.DMA,
        pltpu.SemaphoreType.DMA,
        pltpu.SemaphoreType.DMA,
    ],
)
def _edge_kernel(u_hbm, ei_hbm, out_hbm, src_v, dst_v, rows_v,
                 acc_sh, sem_g0, sem_g1, sem_i0, sem_i1):
    cid = lax.axis_index("c")
    sid = lax.axis_index("s")
    wid = cid * 16 + sid
    r0 = wid * TILE_EROWS
    gsems = [sem_g0, sem_g1]
    isems = [sem_i0, sem_i1]

    def _fetch_idx(row0, slot, n, isem):
        for j in range(n):
            pltpu.async_copy(ei_hbm.at[0, pl.ds((row0 + j) * 128, 128)],
                             src_v.at[slot, j], isem)
            pltpu.async_copy(ei_hbm.at[1, pl.ds((row0 + j) * 128, 128)],
                             dst_v.at[slot, j], isem)

    def _wait_idx(slot, n, isem):
        for j in range(n):
            pltpu.make_async_copy(ei_hbm.at[0, pl.ds(0, 128)],
                                  src_v.at[slot, j], isem).wait()
            pltpu.make_async_copy(ei_hbm.at[1, pl.ds(0, 128)],
                                  dst_v.at[slot, j], isem).wait()

    def _do_windows(slot, n):
        pltpu.async_copy(u_hbm.at[src_v.at[slot, 0]], rows_v.at[0], gsems[0])
        for j in range(1, n):
            b = j % 2
            pltpu.async_copy(u_hbm.at[src_v.at[slot, j]], rows_v.at[b],
                             gsems[b])
            pb = 1 - b
            pltpu.make_async_copy(u_hbm.at[src_v.at[slot, j - 1]],
                                  rows_v.at[pb], gsems[pb]).wait()
            pltpu.sync_copy(rows_v.at[pb],
                            acc_sh.at[dst_v.at[slot, j - 1]], add=True)
        lb = (n - 1) % 2
        pltpu.make_async_copy(u_hbm.at[src_v.at[slot, n - 1]],
                              rows_v.at[lb], gsems[lb]).wait()
        pltpu.sync_copy(rows_v.at[lb], acc_sh.at[dst_v.at[slot, n - 1]],
                        add=True)

    @pl.when(wid < 31)
    def _():
        _fetch_idx(r0, 0, SW, sem_i0)

    @pl.when(wid == 31)
    def _():
        _fetch_idx(r0, 0, SW, sem_i0)
        _fetch_idx(r0 + SW, 1, LAST_TAIL, sem_i1)

    _zero_rows(rows_v.at[0], 128)
    base = sid * ROWS_PER_TILE
    for k in range(ROWS_PER_TILE // 128):
        pltpu.sync_copy(rows_v.at[0], acc_sh.at[pl.ds(base + k * 128, 128)])
    plsc.subcore_barrier()

    def _do_sw(sw, slot):
        nslot = 1 - slot
        nr0 = r0 + jnp.where(sw + 1 < N_SW, (sw + 1) * SW, 0)
        _fetch_idx(nr0, nslot, SW, isems[nslot])
        _wait_idx(slot, SW, isems[slot])
        _do_windows(slot, SW)

    @pl.when(wid < 31)
    def _():
        @pl.loop(0, N_SW - 1, step=2)
        def _(sw):
            _do_sw(sw, 0)
            _do_sw(sw + 1, 1)

        _do_sw(N_SW - 1, 0)
        _wait_idx(1, SW, sem_i1)

    @pl.when(wid == 31)
    def _():
        _wait_idx(0, SW, sem_i0)
        _do_windows(0, SW)
        _wait_idx(1, LAST_TAIL, sem_i1)
        _do_windows(1, LAST_TAIL)

    plsc.subcore_barrier()
    pltpu.sync_copy(acc_sh.at[pl.ds(base, ROWS_PER_TILE)],
                    out_hbm.at[cid, pl.ds(base, ROWS_PER_TILE)])


BM = 1024
GRID = N_PAD // BM
_HI = lax.Precision.DEFAULT


def _g(deg_ref):
    return lax.rsqrt(deg_ref[0, :] + deg_ref[1, :] + 1.0)[:, None]


def _proj_body(x_ref, win_ref, bin_ref, w1_ref, deg_ref, o_ref):
    w = jnp.dot(win_ref[...], w1_ref[...], preferred_element_type=jnp.float32,
                precision=_HI)
    c = jnp.dot(bin_ref[...], w1_ref[...], preferred_element_type=jnp.float32,
                precision=_HI)
    o_ref[...] = _g(deg_ref) * (
        jnp.dot(x_ref[...], w, preferred_element_type=jnp.float32,
                precision=_HI) + c)


_proj = pl.pallas_call(
    _proj_body,
    grid=(GRID,),
    in_specs=[pl.BlockSpec((BM, F), lambda r: (r, 0)),
              pl.BlockSpec((F, F), lambda r: (0, 0)),
              pl.BlockSpec((1, F), lambda r: (0, 0)),
              pl.BlockSpec((F, F), lambda r: (0, 0)),
              pl.BlockSpec((2, BM), lambda r: (0, r))],
    out_specs=pl.BlockSpec((BM, F), lambda r: (r, 0)),
    out_shape=jax.ShapeDtypeStruct((N_PAD, F), jnp.float32),
)


def _layer_body(sa_ref, sb_ref, u_ref, deg_ref, b_ref, w_ref, o_ref):
    g = _g(deg_ref)
    h = jnp.maximum(
        g * (sa_ref[0] + sb_ref[0] + u_ref[...]) + b_ref[...], 0.0)
    o_ref[...] = g * jnp.dot(h, w_ref[...], preferred_element_type=jnp.float32,
                             precision=_HI)


_layer = pl.pallas_call(
    _layer_body,
    grid=(GRID,),
    in_specs=[pl.BlockSpec((1, BM, F), lambda r: (0, r, 0)),
              pl.BlockSpec((1, BM, F), lambda r: (1, r, 0)),
              pl.BlockSpec((BM, F), lambda r: (r, 0)),
              pl.BlockSpec((2, BM), lambda r: (0, r)),
              pl.BlockSpec((1, F), lambda r: (0, 0)),
              pl.BlockSpec((F, F), lambda r: (0, 0))],
    out_specs=pl.BlockSpec((BM, F), lambda r: (r, 0)),
    out_shape=jax.ShapeDtypeStruct((N_PAD, F), jnp.float32),
)


def _final_body(sa_ref, sb_ref, u_ref, deg_ref, b_ref, o_ref):
    g = _g(deg_ref)
    o_ref[...] = jnp.maximum(
        g * (sa_ref[0] + sb_ref[0] + u_ref[...]) + b_ref[...], 0.0)


_final = pl.pallas_call(
    _final_body,
    grid=(GRID,),
    in_specs=[pl.BlockSpec((1, BM, F), lambda r: (0, r, 0)),
              pl.BlockSpec((1, BM, F), lambda r: (1, r, 0)),
              pl.BlockSpec((BM, F), lambda r: (r, 0)),
              pl.BlockSpec((2, BM), lambda r: (0, r)),
              pl.BlockSpec((1, F), lambda r: (0, 0))],
    out_specs=pl.BlockSpec((BM, F), lambda r: (r, 0)),
    out_shape=jax.ShapeDtypeStruct((N, F), jnp.float32),
)


def kernel(x, edge_index, W_in, b_in, W1, b1, W2, b2):
    ei = edge_index.astype(jnp.int32)

    deg = _deg_kernel(ei)
    u1 = _proj(x, W_in, b_in.reshape(1, F), W1, deg)
    s1 = _edge_kernel(u1, ei)
    u2 = _layer(s1, s1, u1, deg, b1.reshape(1, F), W2)
    s2 = _edge_kernel(u2, ei)
    return _final(s2, s2, u2, deg, b2.reshape(1, F))


# speedup vs baseline: 1.0781x; 1.0183x over previous
import functools

import jax
import jax.numpy as jnp
import numpy as np
from jax import lax
from jax.experimental import pallas as pl
from jax.experimental.pallas import tpu as pltpu
from jax.experimental.pallas import tpu_sc as plsc

N = 10000
F = 128
E = 320000
N_PAD = 10240
ROWS_PER_TILE = N_PAD // 16
E_ROWS = 2560
E_PAD = E_ROWS * 128
TILE_EROWS = E_ROWS // 32
SW = 16
N_SW = TILE_EROWS // SW

R_TOTAL = E // 128
LAST_ROWS = R_TOTAL - 31 * TILE_EROWS
LAST_TAIL = LAST_ROWS - SW

_mesh = plsc.VectorSubcoreMesh(core_axis_name="c", subcore_axis_name="s")


def _zero_rows(buf, nrows):
    @pl.loop(0, nrows)
    def _(i):
        for j in range(8):
            buf[i, pl.ds(j * 16, 16)] = jnp.zeros((16,), jnp.float32)


@functools.partial(
    pl.kernel,
    out_type=jax.ShapeDtypeStruct((2, N_PAD), jnp.float32),
    mesh=_mesh,
    scratch_types=[
        pltpu.VMEM((TILE_EROWS, 128), jnp.int32),
        pltpu.VMEM((2, 128), jnp.float32),
        pltpu.VMEM_SHARED((N_PAD,), jnp.float32),
        pltpu.SemaphoreType.DMA,
    ],
)
def _deg_kernel(ei_hbm, out_hbm, dst_v, const_v, acc_sh, sem_i):
    cid = lax.axis_index("c")
    sid = lax.axis_index("s")
    wid = cid * 16 + sid
    r0 = wid * TILE_EROWS
    nw = jnp.where(wid < 31, TILE_EROWS, LAST_ROWS)

    @pl.loop(0, TILE_EROWS)
    def _(w):
        @pl.when(w < nw)
        def _():
            pltpu.async_copy(ei_hbm.at[1, pl.ds((r0 + w) * 128, 128)],
                             dst_v.at[w], sem_i)

    for j in range(8):
        const_v[0, pl.ds(j * 16, 16)] = jnp.ones((16,), jnp.float32)
        const_v[1, pl.ds(j * 16, 16)] = jnp.zeros((16,), jnp.float32)
    base = sid * ROWS_PER_TILE
    for k in range(ROWS_PER_TILE // 128):
        pltpu.sync_copy(const_v.at[1], acc_sh.at[pl.ds(base + k * 128, 128)])

    @pl.loop(0, TILE_EROWS)
    def _(w):
        @pl.when(w < nw)
        def _():
            pltpu.make_async_copy(ei_hbm.at[1, pl.ds(0, 128)], dst_v.at[w],
                                  sem_i).wait()

    plsc.subcore_barrier()

    @pl.loop(0, TILE_EROWS)
    def _(w):
        @pl.when(w < nw)
        def _():
            pltpu.async_copy(const_v.at[0], acc_sh.at[dst_v.at[w]], sem_i,
                             add=True)

    @pl.loop(0, TILE_EROWS)
    def _(w):
        @pl.when(w < nw)
        def _():
            pltpu.make_async_copy(const_v.at[0], acc_sh.at[dst_v.at[w]],
                                  sem_i).wait()

    plsc.subcore_barrier()
    pltpu.sync_copy(acc_sh.at[pl.ds(base, ROWS_PER_TILE)],
                    out_hbm.at[cid, pl.ds(base, ROWS_PER_TILE)])


@functools.partial(
    pl.kernel,
    out_type=jax.ShapeDtypeStruct((2, N_PAD, F), jnp.float32),
    mesh=_mesh,
    scratch_types=[
        pltpu.VMEM((2, SW, 128), jnp.int32),
        pltpu.VMEM((2, SW, 128), jnp.int32),
        pltpu.VMEM((2, 128, F), jnp.float32),
        pltpu.VMEM_SHARED((N_PAD, F), jnp.float32),
        pltpu.SemaphoreType.DMA,
        pltpu.SemaphoreType.DMA,
        pltpu.SemaphoreType.DMA,
        pltpu.SemaphoreType.DMA,
        pltpu.SemaphoreType.DMA,
        pltpu.SemaphoreType.DMA,
    ],
)
def _edge_kernel(u_hbm, ei_hbm, out_hbm, src_v, dst_v, rows_v,
                 acc_sh, sem_g0, sem_g1, sem_i0, sem_i1, sem_s0, sem_s1):
    cid = lax.axis_index("c")
    sid = lax.axis_index("s")
    wid = cid * 16 + sid
    r0 = wid * TILE_EROWS
    gsems = [sem_g0, sem_g1]
    isems = [sem_i0, sem_i1]
    ssems = [sem_s0, sem_s1]

    def _fetch_idx(row0, slot, n, isem):
        for j in range(n):
            pltpu.async_copy(ei_hbm.at[0, pl.ds((row0 + j) * 128, 128)],
                             src_v.at[slot, j], isem)
            pltpu.async_copy(ei_hbm.at[1, pl.ds((row0 + j) * 128, 128)],
                             dst_v.at[slot, j], isem)

    def _wait_idx(slot, n, isem):
        for j in range(n):
            pltpu.make_async_copy(ei_hbm.at[0, pl.ds(0, 128)],
                                  src_v.at[slot, j], isem).wait()
            pltpu.make_async_copy(ei_hbm.at[1, pl.ds(0, 128)],
                                  dst_v.at[slot, j], isem).wait()

    def _do_windows(slot, n):
        pltpu.async_copy(u_hbm.at[src_v.at[slot, 0]], rows_v.at[0], gsems[0])
        for j in range(1, n):
            b = j % 2
            if j >= 2:
                pltpu.make_async_copy(rows_v.at[b],
                                      acc_sh.at[dst_v.at[slot, j - 2]],
                                      ssems[b]).wait()
            pltpu.async_copy(u_hbm.at[src_v.at[slot, j]], rows_v.at[b],
                             gsems[b])
            pb = 1 - b
            pltpu.make_async_copy(u_hbm.at[src_v.at[slot, j - 1]],
                                  rows_v.at[pb], gsems[pb]).wait()
            pltpu.async_copy(rows_v.at[pb],
                             acc_sh.at[dst_v.at[slot, j - 1]], ssems[pb],
                             add=True)
        lb = (n - 1) % 2
        pltpu.make_async_copy(u_hbm.at[src_v.at[slot, n - 1]],
                              rows_v.at[lb], gsems[lb]).wait()
        pltpu.async_copy(rows_v.at[lb], acc_sh.at[dst_v.at[slot, n - 1]],
                         ssems[lb], add=True)
        pltpu.make_async_copy(rows_v.at[1 - lb],
                              acc_sh.at[dst_v.at[slot, n - 2]],
                              ssems[1 - lb]).wait()
        pltpu.make_async_copy(rows_v.at[lb],
                              acc_sh.at[dst_v.at[slot, n - 1]],
                              ssems[lb]).wait()

    @pl.when(wid < 31)
    def _():
        _fetch_idx(r0, 0, SW, sem_i0)

    @pl.when(wid == 31)
    def _():
        _fetch_idx(r0, 0, SW, sem_i0)
        _fetch_idx(r0 + SW, 1, LAST_TAIL, sem_i1)

    _zero_rows(rows_v.at[0], 128)
    base = sid * ROWS_PER_TILE
    for k in range(ROWS_PER_TILE // 128):
        pltpu.sync_copy(rows_v.at[0], acc_sh.at[pl.ds(base + k * 128, 128)])
    plsc.subcore_barrier()

    def _do_sw(sw, slot):
        nslot = 1 - slot
        nr0 = r0 + jnp.where(sw + 1 < N_SW, (sw + 1) * SW, 0)
        _fetch_idx(nr0, nslot, SW, isems[nslot])
        _wait_idx(slot, SW, isems[slot])
        _do_windows(slot, SW)

    @pl.when(wid < 31)
    def _():
        @pl.loop(0, N_SW - 1, step=2)
        def _(sw):
            _do_sw(sw, 0)
            _do_sw(sw + 1, 1)

        _do_sw(N_SW - 1, 0)
        _wait_idx(1, SW, sem_i1)

    @pl.when(wid == 31)
    def _():
        _wait_idx(0, SW, sem_i0)
        _do_windows(0, SW)
        _wait_idx(1, LAST_TAIL, sem_i1)
        _do_windows(1, LAST_TAIL)

    plsc.subcore_barrier()
    pltpu.sync_copy(acc_sh.at[pl.ds(base, ROWS_PER_TILE)],
                    out_hbm.at[cid, pl.ds(base, ROWS_PER_TILE)])


BM = 1024
GRID = N_PAD // BM
_HI = lax.Precision.DEFAULT


def _g(deg_ref):
    return lax.rsqrt(deg_ref[0, :] + deg_ref[1, :] + 1.0)[:, None]


def _proj_body(x_ref, win_ref, bin_ref, w1_ref, deg_ref, o_ref):
    w = jnp.dot(win_ref[...], w1_ref[...], preferred_element_type=jnp.float32,
                precision=_HI)
    c = jnp.dot(bin_ref[...], w1_ref[...], preferred_element_type=jnp.float32,
                precision=_HI)
    o_ref[...] = _g(deg_ref) * (
        jnp.dot(x_ref[...], w, preferred_element_type=jnp.float32,
                precision=_HI) + c)


_proj = pl.pallas_call(
    _proj_body,
    grid=(GRID,),
    in_specs=[pl.BlockSpec((BM, F), lambda r: (r, 0)),
              pl.BlockSpec((F, F), lambda r: (0, 0)),
              pl.BlockSpec((1, F), lambda r: (0, 0)),
              pl.BlockSpec((F, F), lambda r: (0, 0)),
              pl.BlockSpec((2, BM), lambda r: (0, r))],
    out_specs=pl.BlockSpec((BM, F), lambda r: (r, 0)),
    out_shape=jax.ShapeDtypeStruct((N_PAD, F), jnp.float32),
)


def _layer_body(sa_ref, sb_ref, u_ref, deg_ref, b_ref, w_ref, o_ref):
    g = _g(deg_ref)
    h = jnp.maximum(
        g * (sa_ref[0] + sb_ref[0] + u_ref[...]) + b_ref[...], 0.0)
    o_ref[...] = g * jnp.dot(h, w_ref[...], preferred_element_type=jnp.float32,
                             precision=_HI)


_layer = pl.pallas_call(
    _layer_body,
    grid=(GRID,),
    in_specs=[pl.BlockSpec((1, BM, F), lambda r: (0, r, 0)),
              pl.BlockSpec((1, BM, F), lambda r: (1, r, 0)),
              pl.BlockSpec((BM, F), lambda r: (r, 0)),
              pl.BlockSpec((2, BM), lambda r: (0, r)),
              pl.BlockSpec((1, F), lambda r: (0, 0)),
              pl.BlockSpec((F, F), lambda r: (0, 0))],
    out_specs=pl.BlockSpec((BM, F), lambda r: (r, 0)),
    out_shape=jax.ShapeDtypeStruct((N_PAD, F), jnp.float32),
)


def _final_body(sa_ref, sb_ref, u_ref, deg_ref, b_ref, o_ref):
    g = _g(deg_ref)
    o_ref[...] = jnp.maximum(
        g * (sa_ref[0] + sb_ref[0] + u_ref[...]) + b_ref[...], 0.0)


_final = pl.pallas_call(
    _final_body,
    grid=(GRID,),
    in_specs=[pl.BlockSpec((1, BM, F), lambda r: (0, r, 0)),
              pl.BlockSpec((1, BM, F), lambda r: (1, r, 0)),
              pl.BlockSpec((BM, F), lambda r: (r, 0)),
              pl.BlockSpec((2, BM), lambda r: (0, r)),
              pl.BlockSpec((1, F), lambda r: (0, 0))],
    out_specs=pl.BlockSpec((BM, F), lambda r: (r, 0)),
    out_shape=jax.ShapeDtypeStruct((N, F), jnp.float32),
)


def kernel(x, edge_index, W_in, b_in, W1, b1, W2, b2):
    ei = edge_index.astype(jnp.int32)

    deg = _deg_kernel(ei)
    u1 = _proj(x, W_in, b_in.reshape(1, F), W1, deg)
    s1 = _edge_kernel(u1, ei)
    u2 = _layer(s1, s1, u1, deg, b1.reshape(1, F), W2)
    s2 = _edge_kernel(u2, ei)
    return _final(s2, s2, u2, deg, b2.reshape(1, F))
